# Initial kernel scaffold; baseline (speedup 1.0000x reference)
import jax
import jax.numpy as jnp
from jax.experimental import pallas as pl


def kernel(x, edge_index, W_self1, W_neigh1, b1, W_self2, W_neigh2, b2):
    raise NotImplementedError("write your pallas kernel here")


# speedup vs baseline: 3.4799x; 3.4799x over previous
import functools

import jax
import jax.numpy as jnp
from jax import lax
from jax.experimental import pallas as pl
from jax.experimental.pallas import tpu as pltpu
from jax.experimental.pallas import tpu_sc as plsc

_N = 10000
_E = 160000

_NC = 2
_NS = 16
_CHUNK = 128
_NCH = 80
_EPAD = _NS * _NCH * _CHUNK
_NPAD = 10112
_RPT = _NPAD // _NS
_DUMMY = _NPAD - 8

_BM = 400
_GRID = _N // _BM


_NST = 4
_NCHS = _NCH // _NST


def _make_sc_aggregate(wc: int):
    mesh = plsc.VectorSubcoreMesh(
        core_axis_name="c", subcore_axis_name="s",
        num_cores=_NC, num_subcores=_NS)
    scratch = [
        pltpu.VMEM_SHARED((_NPAD, wc), jnp.float32),
        pltpu.VMEM((_NCHS, _CHUNK), jnp.int32),
        pltpu.VMEM((_NCHS, _CHUNK), jnp.int32),
        pltpu.VMEM((_CHUNK, wc), jnp.float32),
        pltpu.SemaphoreType.DMA,
    ]

    def body(z_agg, table, src3, dst3, agg_out,
             agg_sh, src_v, dst_v, rows_v, sem):
        cid = lax.axis_index("c")
        sid = lax.axis_index("s")
        row0 = sid * _RPT

        pltpu.sync_copy(z_agg, agg_sh.at[pl.ds(row0, _RPT), :])
        plsc.subcore_barrier()

        for q in range(_NST):
            pltpu.sync_copy(src3.at[cid, sid, q], src_v)
            pltpu.sync_copy(dst3.at[sid, q], dst_v)

            @pl.loop(0, _NCHS)
            def _(j):
                pltpu.async_copy(table.at[src_v.at[j]], rows_v, sem).wait()
                pltpu.sync_copy(rows_v, agg_sh.at[dst_v.at[j]], add=True)

        plsc.subcore_barrier()
        pltpu.sync_copy(agg_sh.at[pl.ds(row0, _RPT), :],
                        agg_out.at[cid, pl.ds(row0, _RPT), :])

    return pl.kernel(
        body,
        out_type=jax.ShapeDtypeStruct((_NC, _NPAD, wc), jnp.float32),
        mesh=mesh, scratch_types=scratch, name=f"sc_agg_w{wc}",
        compiler_params=pltpu.CompilerParams(use_tc_tiling_on_sc=False))


_sc_agg_l1 = _make_sc_aggregate(144)
_sc_agg_l2 = _make_sc_aggregate(32)


def _rowspec(w):
    return pl.BlockSpec((_BM, w), lambda i: (i, 0))


def _full(shape):
    return pl.BlockSpec(shape, lambda i: (0, 0))


def _tc1_body(x_ref, a0_ref, a1_ref, deg_ref, ws_ref, wn0_ref, wn1_ref,
              b_ref, o_ref):
    r = 1.0 / jnp.maximum(deg_ref[:, 0:1], 1.0)
    acc = jnp.dot(x_ref[...], ws_ref[...], preferred_element_type=jnp.float32)
    acc += jnp.dot(a0_ref[...] * r, wn0_ref[...],
                   preferred_element_type=jnp.float32)
    acc += jnp.dot(a1_ref[...] * r, wn1_ref[...],
                   preferred_element_type=jnp.float32)
    o_ref[...] = jnp.maximum(acc + b_ref[...], 0.0)


_tc_layer1 = pl.pallas_call(
    _tc1_body,
    grid=(_GRID,),
    in_specs=[_rowspec(256), _rowspec(128), _rowspec(128), _rowspec(16),
              _full((256, 256)), _full((128, 256)), _full((128, 256)),
              _full((1, 256))],
    out_specs=_rowspec(256),
    out_shape=jax.ShapeDtypeStruct((_N, 256), jnp.float32),
)


def _tc2_body(h_ref, w_ref, b_ref, o_ref):
    o_ref[...] = jnp.dot(h_ref[...], w_ref[...],
                         preferred_element_type=jnp.float32) + b_ref[...]


_tc_layer2 = pl.pallas_call(
    _tc2_body,
    grid=(_GRID,),
    in_specs=[_rowspec(256), _full((256, 128)), _full((1, 128))],
    out_specs=_rowspec(128),
    out_shape=jax.ShapeDtypeStruct((_N, 128), jnp.float32),
)


def _tcf_body(hsw_ref, a0_ref, a1_ref, deg_ref, o_ref):
    r = 1.0 / jnp.maximum(deg_ref[:, 0:1], 1.0)
    o_ref[...] = hsw_ref[:, :64] + jnp.concatenate(
        [a0_ref[...] * r, a1_ref[...] * r], axis=1)


_tc_final = pl.pallas_call(
    _tcf_body,
    grid=(_GRID,),
    in_specs=[_rowspec(128), _rowspec(32), _rowspec(32), _rowspec(16)],
    out_specs=_rowspec(64),
    out_shape=jax.ShapeDtypeStruct((_N, 64), jnp.float32),
)


def kernel(x, edge_index, W_self1, W_neigh1, b1, W_self2, W_neigh2, b2):
    src = edge_index[0].astype(jnp.int32)
    dst = edge_index[1].astype(jnp.int32)
    pad = _EPAD - _E
    srcp = jnp.concatenate([src, jnp.zeros((pad,), jnp.int32)])
    dstp = jnp.concatenate([dst, jnp.full((pad,), _DUMMY, jnp.int32)])
    src3 = jnp.stack([srcp, srcp + _N]).reshape(_NC, _NS, _NST, _NCHS, _CHUNK)
    dst3 = dstp.reshape(_NS, _NST, _NCHS, _CHUNK)

    xh = jnp.concatenate([x[:, :128], x[:, 128:]], axis=0)
    table1 = jnp.concatenate(
        [xh, jnp.ones((2 * _N, 16), jnp.float32)], axis=1)
    z1 = jnp.zeros((_RPT, 144), jnp.float32)
    agg1 = _sc_agg_l1(z1, table1, src3, dst3)
    a10 = agg1[0, :, :128]
    a11 = agg1[1, :, :128]
    deg16 = agg1[0, :, 128:]

    h = _tc_layer1(x, a10, a11, deg16, W_self1,
                   W_neigh1[:128], W_neigh1[128:],
                   b1.reshape(1, 256))

    wcat = jnp.concatenate([W_self2, W_neigh2], axis=1)
    bcat = jnp.concatenate([b2, jnp.zeros((64,), jnp.float32)]).reshape(1, 128)
    hsw = _tc_layer2(h, wcat, bcat)

    table2 = jnp.concatenate([hsw[:, 64:96], hsw[:, 96:128]], axis=0)
    z2 = jnp.zeros((_RPT, 32), jnp.float32)
    agg2 = _sc_agg_l2(z2, table2, src3, dst3)

    return _tc_final(hsw, agg2[0], agg2[1], deg16)


# speedup vs baseline: 4.8034x; 1.3803x over previous
import jax
import jax.numpy as jnp
from jax import lax
from jax.experimental import pallas as pl
from jax.experimental.pallas import tpu as pltpu
from jax.experimental.pallas import tpu_sc as plsc

_N = 10000
_E = 160000

_NC = 2
_NS = 16
_CHUNK = 128
_NCH = 80
_EPAD = _NS * _NCH * _CHUNK
_NPAD = 10112
_RPT = _NPAD // _NS
_DUMMY = _NPAD - 8
_NST = 4
_NCHS = _NCH // _NST

_BM = 400
_GRID = _N // _BM


def _make_sc_aggregate(wc: int, with_deg: bool):
    mesh = plsc.VectorSubcoreMesh(
        core_axis_name="c", subcore_axis_name="s",
        num_cores=_NC, num_subcores=_NS)
    agg_t = jax.ShapeDtypeStruct((_NC, _NPAD, wc), jnp.float32)
    out_type = ([agg_t, jax.ShapeDtypeStruct((_NC, _NPAD, 16), jnp.float32)]
                if with_deg else agg_t)
    scratch = [
        pltpu.VMEM_SHARED((_NPAD, wc), jnp.float32),
        pltpu.VMEM((_NCHS, _CHUNK), jnp.int32),
        pltpu.VMEM((_NCHS, _CHUNK), jnp.int32),
        pltpu.VMEM((_CHUNK, wc), jnp.float32),
        pltpu.VMEM((_CHUNK, wc), jnp.float32),
        pltpu.SemaphoreType.DMA,
        pltpu.SemaphoreType.DMA,
    ]
    if with_deg:
        scratch += [
            pltpu.VMEM_SHARED((_NPAD, 16), jnp.float32),
            pltpu.VMEM((_CHUNK, 16), jnp.float32),
        ]

    def body(*args):
        if with_deg:
            (z_agg, z_deg, ones, table, src3, dst3, agg_out, deg_out,
             agg_sh, src_v, dst_v, buf_a, buf_b, sem_a, sem_b,
             deg_sh, ones_v) = args
        else:
            (z_agg, table, src3, dst3, agg_out,
             agg_sh, src_v, dst_v, buf_a, buf_b, sem_a, sem_b) = args
        cid = lax.axis_index("c")
        sid = lax.axis_index("s")
        row0 = sid * _RPT

        pltpu.sync_copy(z_agg, agg_sh.at[pl.ds(row0, _RPT), :])
        if with_deg:
            pltpu.sync_copy(z_deg, deg_sh.at[pl.ds(row0, _RPT), :])
            pltpu.sync_copy(ones, ones_v)
        plsc.subcore_barrier()

        def fire(j, buf, sem):
            pltpu.async_copy(table.at[src_v.at[j]], buf, sem)

        def drain(buf, sem):
            pltpu.make_async_copy(table.at[src_v.at[0]], buf, sem).wait()

        def scat(j, buf):
            pltpu.sync_copy(buf, agg_sh.at[dst_v.at[j]], add=True)

        for q in range(_NST):
            pltpu.sync_copy(src3.at[cid, sid, q], src_v)
            pltpu.sync_copy(dst3.at[sid, q], dst_v)

            fire(0, buf_a, sem_a)

            @pl.loop(0, _NCHS // 2 - 1)
            def _(p):
                fire(2 * p + 1, buf_b, sem_b)
                drain(buf_a, sem_a)
                scat(2 * p, buf_a)
                fire(2 * p + 2, buf_a, sem_a)
                drain(buf_b, sem_b)
                scat(2 * p + 1, buf_b)

            fire(_NCHS - 1, buf_b, sem_b)
            drain(buf_a, sem_a)
            scat(_NCHS - 2, buf_a)
            drain(buf_b, sem_b)
            scat(_NCHS - 1, buf_b)

        if with_deg:
            for q in range(_NST // _NC):
                pltpu.sync_copy(dst3.at[sid, (_NST // _NC) * cid + q], dst_v)

                @pl.loop(0, _NCHS)
                def _(j):
                    pltpu.sync_copy(ones_v, deg_sh.at[dst_v.at[j]], add=True)

        plsc.subcore_barrier()
        pltpu.sync_copy(agg_sh.at[pl.ds(row0, _RPT), :],
                        agg_out.at[cid, pl.ds(row0, _RPT), :])
        if with_deg:
            pltpu.sync_copy(deg_sh.at[pl.ds(row0, _RPT), :],
                            deg_out.at[cid, pl.ds(row0, _RPT), :])

    return pl.kernel(
        body, out_type=out_type, mesh=mesh, scratch_types=scratch,
        name=f"sc_agg_w{wc}",
        compiler_params=pltpu.CompilerParams(use_tc_tiling_on_sc=False))


_sc_agg_l1 = _make_sc_aggregate(128, True)
_sc_agg_l2 = _make_sc_aggregate(32, False)


def _rowspec(w):
    return pl.BlockSpec((_BM, w), lambda i: (i, 0))


def _pairspec(w):
    return pl.BlockSpec((_NC, _BM, w), lambda i: (0, i, 0))


def _full(shape):
    return pl.BlockSpec(shape, lambda i: (0, 0))


def _tc1_body(x_ref, agg_ref, deg_ref, ws_ref, wn0_ref, wn1_ref,
              b_ref, o_ref):
    deg = deg_ref[0, :, 0:1] + deg_ref[1, :, 0:1]
    r = 1.0 / jnp.maximum(deg, 1.0)
    acc = jnp.dot(x_ref[...], ws_ref[...], preferred_element_type=jnp.float32)
    acc += jnp.dot(agg_ref[0] * r, wn0_ref[...],
                   preferred_element_type=jnp.float32)
    acc += jnp.dot(agg_ref[1] * r, wn1_ref[...],
                   preferred_element_type=jnp.float32)
    o_ref[...] = jnp.maximum(acc + b_ref[...], 0.0)


_tc_layer1 = pl.pallas_call(
    _tc1_body,
    grid=(_GRID,),
    in_specs=[_rowspec(256), _pairspec(128), _pairspec(16),
              _full((256, 256)), _full((128, 256)), _full((128, 256)),
              _full((1, 256))],
    out_specs=_rowspec(256),
    out_shape=jax.ShapeDtypeStruct((_N, 256), jnp.float32),
)


def _tc2_body(h_ref, ws_ref, wn_ref, b_ref, hs_ref, hw_ref):
    h = h_ref[...]
    hs_ref[...] = jnp.dot(h, ws_ref[...],
                          preferred_element_type=jnp.float32) + b_ref[...]
    hw_ref[...] = jnp.dot(h, wn_ref[...], preferred_element_type=jnp.float32)


_tc_layer2 = pl.pallas_call(
    _tc2_body,
    grid=(_GRID,),
    in_specs=[_rowspec(256), _full((256, 64)), _full((256, 64)),
              _full((1, 64))],
    out_specs=[_rowspec(64), _rowspec(64)],
    out_shape=[jax.ShapeDtypeStruct((_N, 64), jnp.float32),
               jax.ShapeDtypeStruct((_N, 64), jnp.float32)],
)


def _tcf_body(hs_ref, agg_ref, deg_ref, o_ref):
    deg = deg_ref[0, :, 0:1] + deg_ref[1, :, 0:1]
    r = 1.0 / jnp.maximum(deg, 1.0)
    o_ref[...] = hs_ref[...] + jnp.concatenate(
        [agg_ref[0] * r, agg_ref[1] * r], axis=1)


_tc_final = pl.pallas_call(
    _tcf_body,
    grid=(_GRID,),
    in_specs=[_rowspec(64), _pairspec(32), _pairspec(16)],
    out_specs=_rowspec(64),
    out_shape=jax.ShapeDtypeStruct((_N, 64), jnp.float32),
)


def kernel(x, edge_index, W_self1, W_neigh1, b1, W_self2, W_neigh2, b2):
    src = edge_index[0].astype(jnp.int32)
    dst = edge_index[1].astype(jnp.int32)
    pad = _EPAD - _E
    srcp = jnp.concatenate([src, jnp.zeros((pad,), jnp.int32)])
    dstp = jnp.concatenate([dst, jnp.full((pad,), _DUMMY, jnp.int32)])
    base = srcp * 2
    src3 = jnp.stack([base, base + 1]).reshape(_NC, _NS, _NST, _NCHS, _CHUNK)
    dst3 = dstp.reshape(_NS, _NST, _NCHS, _CHUNK)

    z1 = jnp.zeros((_RPT, 128), jnp.float32)
    zd = jnp.zeros((_RPT, 16), jnp.float32)
    ones = jnp.ones((_CHUNK, 16), jnp.float32)
    table1 = x.reshape(2 * _N, 128)
    agg1, deg = _sc_agg_l1(z1, zd, ones, table1, src3, dst3)

    h = _tc_layer1(x, agg1, deg, W_self1,
                   W_neigh1[:128], W_neigh1[128:],
                   b1.reshape(1, 256))

    hs, hw = _tc_layer2(h, W_self2, W_neigh2, b2.reshape(1, 64))

    z2 = jnp.zeros((_RPT, 32), jnp.float32)
    table2 = hw.reshape(2 * _N, 32)
    agg2 = _sc_agg_l2(z2, table2, src3, dst3)

    return _tc_final(hs, agg2, deg)


# speedup vs baseline: 4.8973x; 1.0195x over previous
import jax
import jax.numpy as jnp
from jax import lax
from jax.experimental import pallas as pl
from jax.experimental.pallas import tpu as pltpu
from jax.experimental.pallas import tpu_sc as plsc

_N = 10000
_E = 160000

_NC = 2
_NS = 16
_CHUNK = 128
_EPAD = 163840
_NPAD = 10112
_RPT = _NPAD // _NS
_DUMMY = _NPAD - 8
_NCHS = 20

_BM = 400
_GRID = _N // _BM


def _make_sc_aggregate(wc: int, n_stages: int, core_split: bool,
                       with_deg: bool):
    mesh = plsc.VectorSubcoreMesh(
        core_axis_name="c", subcore_axis_name="s",
        num_cores=_NC, num_subcores=_NS)
    agg_t = jax.ShapeDtypeStruct((_NC, _NPAD, wc), jnp.float32)
    out_type = ([agg_t, jax.ShapeDtypeStruct((_NC, _NPAD, 16), jnp.float32)]
                if with_deg else agg_t)
    scratch = [
        pltpu.VMEM_SHARED((_NPAD, wc), jnp.float32),
        pltpu.VMEM((_NCHS, _CHUNK), jnp.int32),
        pltpu.VMEM((_NCHS, _CHUNK), jnp.int32),
        pltpu.VMEM((_CHUNK, wc), jnp.float32),
        pltpu.VMEM((_CHUNK, wc), jnp.float32),
        pltpu.SemaphoreType.DMA,
        pltpu.SemaphoreType.DMA,
    ]
    if with_deg:
        scratch += [
            pltpu.VMEM_SHARED((_NPAD, 16), jnp.float32),
            pltpu.VMEM((_CHUNK, 16), jnp.float32),
        ]

    def body(*args):
        if with_deg:
            (z_agg, z_deg, ones, table, src, dst, agg_out, deg_out,
             agg_sh, src_v, dst_v, buf_a, buf_b, sem_a, sem_b,
             deg_sh, ones_v) = args
        else:
            (z_agg, table, src, dst, agg_out,
             agg_sh, src_v, dst_v, buf_a, buf_b, sem_a, sem_b) = args
        cid = lax.axis_index("c")
        sid = lax.axis_index("s")
        row0 = sid * _RPT

        pltpu.sync_copy(z_agg, agg_sh.at[pl.ds(row0, _RPT), :])
        if with_deg:
            pltpu.sync_copy(z_deg, deg_sh.at[pl.ds(row0, _RPT), :])
            pltpu.sync_copy(ones, ones_v)
        plsc.subcore_barrier()

        def fire(j, buf, sem):
            pltpu.async_copy(table.at[src_v.at[j]], buf, sem)

        def drain(buf, sem):
            pltpu.make_async_copy(table.at[src_v.at[0]], buf, sem).wait()

        def scat(j, buf):
            pltpu.sync_copy(buf, agg_sh.at[dst_v.at[j]], add=True)

        for q in range(n_stages):
            pltpu.sync_copy(src.at[cid, sid, q], src_v)
            if core_split:
                pltpu.sync_copy(dst.at[cid, sid, q], dst_v)
            else:
                pltpu.sync_copy(dst.at[sid, q], dst_v)

            fire(0, buf_a, sem_a)

            @pl.loop(0, _NCHS // 2 - 1)
            def _(p):
                fire(2 * p + 1, buf_b, sem_b)
                drain(buf_a, sem_a)
                scat(2 * p, buf_a)
                fire(2 * p + 2, buf_a, sem_a)
                drain(buf_b, sem_b)
                scat(2 * p + 1, buf_b)

            fire(_NCHS - 1, buf_b, sem_b)
            drain(buf_a, sem_a)
            scat(_NCHS - 2, buf_a)
            drain(buf_b, sem_b)
            scat(_NCHS - 1, buf_b)

        if with_deg:
            for q in range(n_stages // _NC):
                pltpu.sync_copy(dst.at[sid, (n_stages // _NC) * cid + q],
                                dst_v)

                @pl.loop(0, _NCHS)
                def _(j):
                    pltpu.sync_copy(ones_v, deg_sh.at[dst_v.at[j]], add=True)

        plsc.subcore_barrier()
        pltpu.sync_copy(agg_sh.at[pl.ds(row0, _RPT), :],
                        agg_out.at[cid, pl.ds(row0, _RPT), :])
        if with_deg:
            pltpu.sync_copy(deg_sh.at[pl.ds(row0, _RPT), :],
                            deg_out.at[cid, pl.ds(row0, _RPT), :])

    return pl.kernel(
        body, out_type=out_type, mesh=mesh, scratch_types=scratch,
        name=f"sc_agg_w{wc}",
        compiler_params=pltpu.CompilerParams(use_tc_tiling_on_sc=False))


_sc_agg_l1 = _make_sc_aggregate(128, 4, False, True)
_sc_agg_l2 = _make_sc_aggregate(64, 2, True, False)


def _rowspec(w):
    return pl.BlockSpec((_BM, w), lambda i: (i, 0))


def _pairspec(w):
    return pl.BlockSpec((_NC, _BM, w), lambda i: (0, i, 0))


def _full(shape):
    return pl.BlockSpec(shape, lambda i: (0, 0))


def _tcs_body(x_ref, ws_ref, b_ref, o_ref):
    o_ref[...] = jnp.dot(x_ref[...], ws_ref[...],
                         preferred_element_type=jnp.float32) + b_ref[...]


_tc_self1 = pl.pallas_call(
    _tcs_body,
    grid=(_GRID,),
    in_specs=[_rowspec(256), _full((256, 256)), _full((1, 256))],
    out_specs=_rowspec(256),
    out_shape=jax.ShapeDtypeStruct((_N, 256), jnp.float32),
)


def _tca_body(xs_ref, agg_ref, deg_ref, wn0_ref, wn1_ref, ws2_ref, wn2_ref,
              b2_ref, hs_ref, hw_ref):
    deg = deg_ref[0, :, 0:1] + deg_ref[1, :, 0:1]
    r = 1.0 / jnp.maximum(deg, 1.0)
    acc = xs_ref[...]
    acc += jnp.dot(agg_ref[0] * r, wn0_ref[...],
                   preferred_element_type=jnp.float32)
    acc += jnp.dot(agg_ref[1] * r, wn1_ref[...],
                   preferred_element_type=jnp.float32)
    h = jnp.maximum(acc, 0.0)
    hs_ref[...] = jnp.dot(h, ws2_ref[...],
                          preferred_element_type=jnp.float32) + b2_ref[...]
    hw_ref[...] = jnp.dot(h, wn2_ref[...], preferred_element_type=jnp.float32)


_tc_mid = pl.pallas_call(
    _tca_body,
    grid=(_GRID,),
    in_specs=[_rowspec(256), _pairspec(128), _pairspec(16),
              _full((128, 256)), _full((128, 256)),
              _full((256, 64)), _full((256, 64)), _full((1, 64))],
    out_specs=[_rowspec(64), _rowspec(64)],
    out_shape=[jax.ShapeDtypeStruct((_N, 64), jnp.float32),
               jax.ShapeDtypeStruct((_N, 64), jnp.float32)],
)


def _tcf_body(hs_ref, agg_ref, deg_ref, o_ref):
    deg = deg_ref[0, :, 0:1] + deg_ref[1, :, 0:1]
    r = 1.0 / jnp.maximum(deg, 1.0)
    o_ref[...] = hs_ref[...] + (agg_ref[0] + agg_ref[1]) * r


_tc_final = pl.pallas_call(
    _tcf_body,
    grid=(_GRID,),
    in_specs=[_rowspec(64), _pairspec(64), _pairspec(16)],
    out_specs=_rowspec(64),
    out_shape=jax.ShapeDtypeStruct((_N, 64), jnp.float32),
)


def kernel(x, edge_index, W_self1, W_neigh1, b1, W_self2, W_neigh2, b2):
    src = edge_index[0].astype(jnp.int32)
    dst = edge_index[1].astype(jnp.int32)
    pad = _EPAD - _E
    srcp = jnp.concatenate([src, jnp.zeros((pad,), jnp.int32)])
    dstp = jnp.concatenate([dst, jnp.full((pad,), _DUMMY, jnp.int32)])
    base = srcp * 2
    src3 = jnp.stack([base, base + 1]).reshape(_NC, _NS, 4, _NCHS, _CHUNK)
    dst3 = dstp.reshape(_NS, 4, _NCHS, _CHUNK)
    src4 = srcp.reshape(_NC, _NS, 2, _NCHS, _CHUNK)
    dst4 = dstp.reshape(_NC, _NS, 2, _NCHS, _CHUNK)

    z1 = jnp.zeros((_RPT, 128), jnp.float32)
    zd = jnp.zeros((_RPT, 16), jnp.float32)
    ones = jnp.ones((_CHUNK, 16), jnp.float32)
    table1 = x.reshape(2 * _N, 128)
    agg1, deg = _sc_agg_l1(z1, zd, ones, table1, src3, dst3)

    xs = _tc_self1(x, W_self1, b1.reshape(1, 256))
    hs, hw = _tc_mid(xs, agg1, deg, W_neigh1[:128], W_neigh1[128:],
                     W_self2, W_neigh2, b2.reshape(1, 64))

    z2 = jnp.zeros((_RPT, 64), jnp.float32)
    agg2 = _sc_agg_l2(z2, hw, src4, dst4)

    return _tc_final(hs, agg2, deg)


# speedup vs baseline: 5.0176x; 1.0246x over previous
import jax
import jax.numpy as jnp
from jax import lax
from jax.experimental import pallas as pl
from jax.experimental.pallas import tpu as pltpu
from jax.experimental.pallas import tpu_sc as plsc

_N = 10000
_E = 160000

_NC = 2
_NS = 16
_CHUNK = 128
_EPAD = 163840
_NPAD = 10112
_RPT = _NPAD // _NS
_DUMMY = _NPAD - 8
_NCHS = 20

_BM = 400
_GRID = _N // _BM


def _make_sc_aggregate(wc: int, n_stages: int, core_split: bool,
                       with_deg: bool, chunk: int, nchs: int, depth: int):
    mesh = plsc.VectorSubcoreMesh(
        core_axis_name="c", subcore_axis_name="s",
        num_cores=_NC, num_subcores=_NS)
    agg_t = jax.ShapeDtypeStruct((_NC, _NPAD, wc), jnp.float32)
    out_type = ([agg_t, jax.ShapeDtypeStruct((_NC, _NPAD, 16), jnp.float32)]
                if with_deg else agg_t)
    scratch = [
        pltpu.VMEM_SHARED((_NPAD, wc), jnp.float32),
        pltpu.VMEM((nchs, chunk), jnp.int32),
        pltpu.VMEM((nchs, chunk), jnp.int32),
    ]
    scratch += [pltpu.VMEM((chunk, wc), jnp.float32) for _ in range(depth)]
    scratch += [pltpu.SemaphoreType.DMA for _ in range(depth)]
    if with_deg:
        scratch += [
            pltpu.VMEM_SHARED((_NPAD, 16), jnp.float32),
            pltpu.VMEM((chunk, 16), jnp.float32),
        ]

    def body(*args):
        if with_deg:
            (z_agg, z_deg, ones, table, src, dst, agg_out, deg_out,
             agg_sh, src_v, dst_v, *rest) = args
            bufs, sems = rest[:depth], rest[depth:2 * depth]
            deg_sh, ones_v = rest[2 * depth:]
        else:
            (z_agg, table, src, dst, agg_out,
             agg_sh, src_v, dst_v, *rest) = args
            bufs, sems = rest[:depth], rest[depth:2 * depth]
        cid = lax.axis_index("c")
        sid = lax.axis_index("s")
        row0 = sid * _RPT

        pltpu.sync_copy(z_agg, agg_sh.at[pl.ds(row0, _RPT), :])
        if with_deg:
            pltpu.sync_copy(z_deg, deg_sh.at[pl.ds(row0, _RPT), :])
            pltpu.sync_copy(ones, ones_v)
        plsc.subcore_barrier()

        def fire(j, k):
            pltpu.async_copy(table.at[src_v.at[j]], bufs[k], sems[k])

        def drain(k):
            pltpu.make_async_copy(
                table.at[src_v.at[0]], bufs[k], sems[k]).wait()

        def scat(j, k, deg_on):
            pltpu.sync_copy(bufs[k], agg_sh.at[dst_v.at[j]], add=True)
            if with_deg:
                @pl.when(deg_on)
                def _():
                    pltpu.sync_copy(ones_v, deg_sh.at[dst_v.at[j]], add=True)

        for q in range(n_stages):
            pltpu.sync_copy(src.at[cid, sid, q], src_v)
            if core_split:
                pltpu.sync_copy(dst.at[cid, sid, q], dst_v)
            else:
                pltpu.sync_copy(dst.at[sid, q], dst_v)
            deg_on = q // (n_stages // _NC) == cid

            for k in range(depth - 1):
                fire(k, k)

            @pl.loop(0, nchs // depth - 1)
            def _(p):
                for k in range(depth):
                    j = depth * p + k
                    fire(j + depth - 1, (k + depth - 1) % depth)
                    drain(k)
                    scat(j, k, deg_on)

            fire(nchs - 1, depth - 1)
            for k in range(depth):
                drain(k)
                scat(nchs - depth + k, k, deg_on)

        plsc.subcore_barrier()
        pltpu.sync_copy(agg_sh.at[pl.ds(row0, _RPT), :],
                        agg_out.at[cid, pl.ds(row0, _RPT), :])
        if with_deg:
            pltpu.sync_copy(deg_sh.at[pl.ds(row0, _RPT), :],
                            deg_out.at[cid, pl.ds(row0, _RPT), :])

    return pl.kernel(
        body, out_type=out_type, mesh=mesh, scratch_types=scratch,
        name=f"sc_agg_w{wc}",
        compiler_params=pltpu.CompilerParams(use_tc_tiling_on_sc=False))


_sc_agg_l1 = _make_sc_aggregate(128, 4, False, True, _CHUNK, _NCHS, 2)
_sc_agg_l2 = _make_sc_aggregate(64, 2, True, False, _CHUNK, _NCHS, 4)


def _rowspec(w):
    return pl.BlockSpec((_BM, w), lambda i: (i, 0))


def _pairspec(w):
    return pl.BlockSpec((_NC, _BM, w), lambda i: (0, i, 0))


def _full(shape):
    return pl.BlockSpec(shape, lambda i: (0, 0))


def _tcs_body(x_ref, ws_ref, b_ref, o_ref):
    o_ref[...] = jnp.dot(x_ref[...], ws_ref[...],
                         preferred_element_type=jnp.float32) + b_ref[...]


_tc_self1 = pl.pallas_call(
    _tcs_body,
    grid=(_GRID,),
    in_specs=[_rowspec(256), _full((256, 256)), _full((1, 256))],
    out_specs=_rowspec(256),
    out_shape=jax.ShapeDtypeStruct((_N, 256), jnp.float32),
)


def _tca_body(xs_ref, agg_ref, deg_ref, wn0_ref, wn1_ref, ws2_ref, wn2_ref,
              b2_ref, hs_ref, hw_ref):
    deg = deg_ref[0, :, 0:1] + deg_ref[1, :, 0:1]
    r = 1.0 / jnp.maximum(deg, 1.0)
    acc = xs_ref[...]
    acc += jnp.dot(agg_ref[0] * r, wn0_ref[...],
                   preferred_element_type=jnp.float32)
    acc += jnp.dot(agg_ref[1] * r, wn1_ref[...],
                   preferred_element_type=jnp.float32)
    h = jnp.maximum(acc, 0.0)
    hs_ref[...] = jnp.dot(h, ws2_ref[...],
                          preferred_element_type=jnp.float32) + b2_ref[...]
    hw_ref[...] = jnp.dot(h, wn2_ref[...], preferred_element_type=jnp.float32)


_tc_mid = pl.pallas_call(
    _tca_body,
    grid=(_GRID,),
    in_specs=[_rowspec(256), _pairspec(128), _pairspec(16),
              _full((128, 256)), _full((128, 256)),
              _full((256, 64)), _full((256, 64)), _full((1, 64))],
    out_specs=[_rowspec(64), _rowspec(64)],
    out_shape=[jax.ShapeDtypeStruct((_N, 64), jnp.float32),
               jax.ShapeDtypeStruct((_N, 64), jnp.float32)],
)


def _tcf_body(hs_ref, agg_ref, deg_ref, o_ref):
    deg = deg_ref[0, :, 0:1] + deg_ref[1, :, 0:1]
    r = 1.0 / jnp.maximum(deg, 1.0)
    o_ref[...] = hs_ref[...] + (agg_ref[0] + agg_ref[1]) * r


_tc_final = pl.pallas_call(
    _tcf_body,
    grid=(_GRID,),
    in_specs=[_rowspec(64), _pairspec(64), _pairspec(16)],
    out_specs=_rowspec(64),
    out_shape=jax.ShapeDtypeStruct((_N, 64), jnp.float32),
)


def kernel(x, edge_index, W_self1, W_neigh1, b1, W_self2, W_neigh2, b2):
    src = edge_index[0].astype(jnp.int32)
    dst = edge_index[1].astype(jnp.int32)
    pad = _EPAD - _E
    srcp = jnp.concatenate([src, jnp.zeros((pad,), jnp.int32)])
    dstp = jnp.concatenate([dst, jnp.full((pad,), _DUMMY, jnp.int32)])
    base = srcp * 2
    src3 = jnp.stack([base, base + 1]).reshape(_NC, _NS, 4, _NCHS, _CHUNK)
    dst3 = dstp.reshape(_NS, 4, _NCHS, _CHUNK)
    src4 = srcp.reshape(_NC, _NS, 2, _NCHS, _CHUNK)
    dst4 = dstp.reshape(_NC, _NS, 2, _NCHS, _CHUNK)

    z1 = jnp.zeros((_RPT, 128), jnp.float32)
    zd = jnp.zeros((_RPT, 16), jnp.float32)
    ones = jnp.ones((_CHUNK, 16), jnp.float32)
    table1 = x.reshape(2 * _N, 128)
    agg1, deg = _sc_agg_l1(z1, zd, ones, table1, src3, dst3)

    xs = _tc_self1(x, W_self1, b1.reshape(1, 256))
    hs, hw = _tc_mid(xs, agg1, deg, W_neigh1[:128], W_neigh1[128:],
                     W_self2, W_neigh2, b2.reshape(1, 64))

    z2 = jnp.zeros((_RPT, 64), jnp.float32)
    agg2 = _sc_agg_l2(z2, hw, src4, dst4)

    return _tc_final(hs, agg2, deg)
